# Initial kernel scaffold; baseline (speedup 1.0000x reference)
import jax
import jax.numpy as jnp
from jax.experimental import pallas as pl


def kernel(vertices, adj_indices, adj_values, emb, gcn_W, gcn_b, gcn_W_last, gcn_b_last, bn_gamma, bn_beta, bn_gamma_last, bn_beta_last, te_W, te_b, mask_w, mask_b):
    raise NotImplementedError("write your pallas kernel here")


# speedup vs baseline: 4.1365x; 4.1365x over previous
import functools

import numpy as np
import jax
import jax.numpy as jnp
from jax import lax
from jax.experimental import pallas as pl
from jax.experimental.pallas import tpu as pltpu
from jax.experimental.pallas import tpu_sc as plsc

N = 10000
D = 128
E = 320000
V = 100000
NHID = 7
EPS = 1e-5
NC, NS = 2, 16
NW = NC * NS
EPT = E // NW
C = 80
NCHUNK = EPT // C
ZR = 624
ZTAIL = N - NS * ZR
GCH = N // 8

_HI = lax.Precision.HIGHEST
_DN = lax.GatherDimensionNumbers(
    offset_dims=(), collapsed_slice_dims=(0,), start_index_map=(0,))


def _bcast_lane(v16, j):
    return lax.gather(v16, jnp.full((16, 1), j, jnp.int32), _DN, (1,),
                      mode=lax.GatherScatterMode.PROMISE_IN_BOUNDS)
_mesh = plsc.VectorSubcoreMesh(core_axis_name="c", subcore_axis_name="s")


def _emb_body(verts, emb, mask2d, x0, w2d, idxb, rowb, mrowb, sem):
    tid = lax.axis_index("c") * NS + lax.axis_index("s")

    nmine = jnp.where(tid < GCH % NW, GCH // NW + 1, GCH // NW)

    def body(i, carry):
        base = pl.multiple_of((tid + i * NW) * 8, 8)
        pltpu.sync_copy(verts.at[pl.ds(base, 8)], idxb)
        pltpu.async_copy(emb.at[idxb], rowb, sem).wait()
        pltpu.sync_copy(rowb, x0.at[pl.ds(base, 8)])
        pltpu.async_copy(mask2d.at[idxb], mrowb, sem).wait()
        pltpu.sync_copy(mrowb, w2d.at[pl.ds(base, 8)])
        return carry

    lax.fori_loop(0, nmine, body, 0)


_emb_gather = functools.partial(
    pl.kernel,
    out_type=(jax.ShapeDtypeStruct((N, D), jnp.float32),
              jax.ShapeDtypeStruct((N, D), jnp.float32)),
    mesh=_mesh,
    scratch_types=[
        pltpu.VMEM((8,), jnp.int32),
        pltpu.VMEM((8, D), jnp.float32),
        pltpu.VMEM((8, D), jnp.float32),
        pltpu.SemaphoreType.DMA,
    ],
)(_emb_body)


def _spmm_body(support, src, dst, vals, zeros, out,
               srcb, dstb, valb, rows, accum, sem):
    cid = lax.axis_index("c")
    sid = lax.axis_index("s")
    tid = cid * NS + sid
    rb = pl.multiple_of(sid * ZR, 8)
    pltpu.sync_copy(zeros.at[pl.ds(rb, ZR)], accum.at[pl.ds(rb, ZR)])

    @pl.when(sid == NS - 1)
    def _():
        pltpu.sync_copy(zeros.at[pl.ds(NS * ZR, ZTAIL)],
                        accum.at[pl.ds(NS * ZR, ZTAIL)])

    plsc.subcore_barrier()
    ebase = tid * EPT

    def chunk(k, carry):
        base = pl.multiple_of(ebase + k * C, 8)
        pltpu.sync_copy(src.at[pl.ds(base, C)], srcb)
        pltpu.sync_copy(dst.at[pl.ds(base, C)], dstb)
        pltpu.sync_copy(vals.at[pl.ds(base, C)], valb)
        pltpu.async_copy(support.at[srcb], rows, sem).wait()

        def grp(g, c2):
            v16 = valb[pl.ds(g * 16, 16)]
            for j in range(16):
                e = g * 16 + j
                vj = _bcast_lane(v16, j)
                for cb in range(D // 16):
                    sl = pl.ds(cb * 16, 16)
                    rows[e, sl] = rows[e, sl] * vj
            return c2

        lax.fori_loop(0, C // 16, grp, 0)
        pltpu.sync_copy(rows, accum.at[dstb], add=True)
        return carry

    lax.fori_loop(0, NCHUNK, chunk, 0)
    plsc.subcore_barrier()
    pltpu.sync_copy(accum.at[pl.ds(rb, ZR)], out.at[cid, pl.ds(rb, ZR)])

    @pl.when(sid == NS - 1)
    def _():
        pltpu.sync_copy(accum.at[pl.ds(NS * ZR, ZTAIL)],
                        out.at[cid, pl.ds(NS * ZR, ZTAIL)])


_spmm = functools.partial(
    pl.kernel,
    out_type=jax.ShapeDtypeStruct((NC, N, D), jnp.float32),
    mesh=_mesh,
    scratch_types=[
        pltpu.VMEM((C,), jnp.int32),
        pltpu.VMEM((C,), jnp.int32),
        pltpu.VMEM((C,), jnp.float32),
        pltpu.VMEM((C, D), jnp.float32),
        pltpu.VMEM_SHARED((N, D), jnp.float32),
        pltpu.SemaphoreType.DMA,
    ],
)(_spmm_body)


def _dot(a, b):
    return jnp.dot(a, b, preferred_element_type=jnp.float32)


def _start_body(x_ref, w_ref, b_ref, tw_ref, tb_ref, sup_ref, gate_ref):
    x = x_ref[...]
    sup_ref[...] = _dot(x, w_ref[...]) + b_ref[...]
    gate_ref[...] = jax.nn.sigmoid(_dot(x, tw_ref[...]) + tb_ref[...])


def _bn_relu(hp0, hp1, gamma, beta):
    h = hp0 + hp1
    mu = jnp.mean(h, axis=0, keepdims=True)
    var = jnp.mean((h - mu) ** 2, axis=0, keepdims=True)
    hn = gamma * (h - mu) / jnp.sqrt(var + EPS) + beta
    return jnp.maximum(hn, 0.0)


def _advance_body(hp_ref, x_ref, gate_ref, gm_ref, bt_ref, w_ref, b_ref,
                  tw_ref, tb_ref, x_out, sup_out, gate_out):
    hr = _bn_relu(hp_ref[0], hp_ref[1], gm_ref[...], bt_ref[...])
    g = gate_ref[...]
    xn = g * hr + (1.0 - g) * x_ref[...]
    x_out[...] = xn
    sup_out[...] = _dot(xn, w_ref[...]) + b_ref[...]
    gate_out[...] = jax.nn.sigmoid(_dot(xn, tw_ref[...]) + tb_ref[...])


def _prelast_body(hp_ref, x_ref, gate_ref, gm_ref, bt_ref, w_ref, b_ref,
                  sup_ref):
    hr = _bn_relu(hp_ref[0], hp_ref[1], gm_ref[...], bt_ref[...])
    g = gate_ref[...]
    xn = g * hr + (1.0 - g) * x_ref[...]
    sup_ref[...] = _dot(xn, w_ref[...]) + b_ref[...]


def _final_body(hp_ref, w_ref, gl_ref, bl_ref, mb_ref, out_ref):
    hr = _bn_relu(hp_ref[0], hp_ref[1], gl_ref[0, 0], bl_ref[0, 0])
    s = jnp.sum(hr * w_ref[...], axis=0, keepdims=True)
    out_ref[...] = jax.nn.sigmoid(s[:, 0:1] + mb_ref[...])


def _sd(*shape):
    return jax.ShapeDtypeStruct(shape, jnp.float32)


_tc_start = pl.pallas_call(_start_body, out_shape=(_sd(N, D), _sd(N, D)))
_tc_advance = pl.pallas_call(
    _advance_body, out_shape=(_sd(N, D), _sd(N, D), _sd(N, D)))
_tc_prelast = pl.pallas_call(_prelast_body, out_shape=_sd(N, D))
_tc_final = pl.pallas_call(_final_body, out_shape=_sd(1, 1))


def kernel(vertices, adj_indices, adj_values, emb, gcn_W, gcn_b, gcn_W_last,
           gcn_b_last, bn_gamma, bn_beta, bn_gamma_last, bn_beta_last,
           te_W, te_b, mask_w, mask_b):
    verts = vertices.astype(jnp.int32)
    adj_idx = adj_indices.astype(jnp.int32)
    w_last = jnp.tile(gcn_W_last, (1, D))
    b_last = jnp.tile(gcn_b_last, (D,)).reshape(1, D)
    mask2d = jnp.tile(mask_w.reshape(-1, 1), (1, D))
    zeros_d = jnp.zeros((N, D), jnp.float32)

    x0, w2d = _emb_gather(verts, emb, mask2d)
    x = x0
    sup, gate = _tc_start(x0, gcn_W[0], gcn_b[0].reshape(1, D),
                          te_W[NHID - 2], te_b[NHID - 2].reshape(1, D))
    for i in range(NHID - 1):
        hp = _spmm(sup, adj_idx[i, 1], adj_idx[i, 0], adj_values[i], zeros_d)
        gm = bn_gamma[i].reshape(1, D)
        bt = bn_beta[i].reshape(1, D)
        if i < NHID - 2:
            x, sup, gate = _tc_advance(
                hp, x, gate, gm, bt,
                gcn_W[i + 1], gcn_b[i + 1].reshape(1, D),
                te_W[i], te_b[i].reshape(1, D))
        else:
            sup = _tc_prelast(hp, x, gate, gm, bt, w_last, b_last)
    hp = _spmm(sup, adj_idx[NHID - 1, 1], adj_idx[NHID - 1, 0],
               adj_values[NHID - 1], zeros_d)
    out = _tc_final(hp, w2d, bn_gamma_last.reshape(1, 1),
                    bn_beta_last.reshape(1, 1), mask_b.reshape(1, 1))
    return out.reshape(1)


# speedup vs baseline: 10.1787x; 2.4607x over previous
import functools

import numpy as np
import jax
import jax.numpy as jnp
from jax import lax
from jax.experimental import pallas as pl
from jax.experimental.pallas import tpu as pltpu
from jax.experimental.pallas import tpu_sc as plsc

N = 10000
D = 128
E = 320000
V = 100000
NHID = 7
EPS = 1e-5
NC, NS = 2, 16
NW = NC * NS
EPT = E // NW
C = 80
NCHUNK = EPT // C
SBN = 5
SBC = NCHUNK // SBN
SB = SBC * C
ZR = 624
ZTAIL = N - NS * ZR
GCH = N // 8

_HI = lax.Precision.HIGHEST
_DN = lax.GatherDimensionNumbers(
    offset_dims=(), collapsed_slice_dims=(0,), start_index_map=(0,))


def _bcast_lane(v16, j):
    return lax.gather(v16, jnp.full((16, 1), j, jnp.int32), _DN, (1,),
                      mode=lax.GatherScatterMode.PROMISE_IN_BOUNDS)
_mesh = plsc.VectorSubcoreMesh(core_axis_name="c", subcore_axis_name="s")


def _emb_body(verts, emb, mask2d, x0, w2d, idxb, rowb, mrowb, sem):
    tid = lax.axis_index("c") * NS + lax.axis_index("s")

    nmine = jnp.where(tid < GCH % NW, GCH // NW + 1, GCH // NW)

    def body(i, carry):
        base = pl.multiple_of((tid + i * NW) * 8, 8)
        pltpu.sync_copy(verts.at[pl.ds(base, 8)], idxb)
        pltpu.async_copy(emb.at[idxb], rowb, sem).wait()
        pltpu.sync_copy(rowb, x0.at[pl.ds(base, 8)])
        pltpu.async_copy(mask2d.at[idxb], mrowb, sem).wait()
        pltpu.sync_copy(mrowb, w2d.at[pl.ds(base, 8)])
        return carry

    lax.fori_loop(0, nmine, body, 0)


_emb_gather = functools.partial(
    pl.kernel,
    out_type=(jax.ShapeDtypeStruct((N, D), jnp.float32),
              jax.ShapeDtypeStruct((N, D), jnp.float32)),
    mesh=_mesh,
    scratch_types=[
        pltpu.VMEM((8,), jnp.int32),
        pltpu.VMEM((8, D), jnp.float32),
        pltpu.VMEM((8, D), jnp.float32),
        pltpu.SemaphoreType.DMA,
    ],
)(_emb_body)


def _spmm_body(support, src, dst4, vals, zeros, out,
               srcv, valv, dstv, rows0, rows1, rows2, accum,
               sg0, sg1, sg2, ss0, ss1, ss2):
    rowsb = (rows0, rows1, rows2)
    sg = (sg0, sg1, sg2)
    ss = (ss0, ss1, ss2)
    cid = lax.axis_index("c")
    sid = lax.axis_index("s")
    tid = cid * NS + sid
    rb = pl.multiple_of(sid * ZR, 8)
    pltpu.sync_copy(zeros.at[pl.ds(rb, ZR)], accum.at[pl.ds(rb, ZR)])

    @pl.when(sid == NS - 1)
    def _():
        pltpu.sync_copy(zeros.at[pl.ds(NS * ZR, ZTAIL)],
                        accum.at[pl.ds(NS * ZR, ZTAIL)])

    def gather_start(k, b):
        pltpu.async_copy(support.at[srcv.at[pl.ds(k * C, C)]], rowsb[b],
                         sg[b])

    def gather_wait(k, b):
        pltpu.make_async_copy(support.at[srcv.at[pl.ds(k * C, C)]],
                              rowsb[b], sg[b]).wait()

    def scale(k, b):
        r = rowsb[b]

        def grp(g, c2):
            v16 = valv[pl.ds(k * C + g * 16, 16)]
            for j in range(16):
                e = g * 16 + j
                vj = _bcast_lane(v16, j)
                for cb in range(D // 16):
                    sl = pl.ds(cb * 16, 16)
                    r[e, sl] = r[e, sl] * vj
            return c2

        lax.fori_loop(0, C // 16, grp, 0)

    def scatter_start(k, b):
        pltpu.async_copy(rowsb[b], accum.at[dstv.at[k]], ss[b], add=True)

    def scatter_wait(k, b):
        pltpu.make_async_copy(rowsb[b], accum.at[dstv.at[k]], ss[b]).wait()

    def step(k, b, b2, refill):
        if refill:
            scatter_wait(k - 1, b2)
            gather_start(k + 2, b2)
        gather_wait(k, b)
        scale(k, b)
        scatter_start(k, b)

    def sblock(s, carry):
        base = pl.multiple_of(tid * EPT + s * SB, 8)
        pltpu.sync_copy(src.at[pl.ds(base, SB)], srcv)
        pltpu.sync_copy(vals.at[pl.ds(base, SB)], valv)
        pltpu.sync_copy(dst4.at[tid, s], dstv)
        gather_start(0, 0)
        gather_start(1, 1)
        gather_start(2, 2)
        step(0, 0, None, False)
        step(1, 1, 0, True)

        def tri(t, c2):
            k = 2 + t * 3
            step(k, 2, 1, True)
            step(k + 1, 0, 2, True)
            step(k + 2, 1, 0, True)
            return c2

        nsteady = (SBC - 5) // 3
        lax.fori_loop(0, nsteady, tri, 0)
        for k in range(2 + nsteady * 3, SBC):
            step(k, k % 3, (k + 2) % 3, k + 2 < SBC)
        for k in range(SBC - 3, SBC):
            scatter_wait(k, k % 3)
        return carry

    plsc.subcore_barrier()
    lax.fori_loop(0, SBN, sblock, 0)

    plsc.subcore_barrier()
    pltpu.sync_copy(accum.at[pl.ds(rb, ZR)], out.at[cid, pl.ds(rb, ZR)])

    @pl.when(sid == NS - 1)
    def _():
        pltpu.sync_copy(accum.at[pl.ds(NS * ZR, ZTAIL)],
                        out.at[cid, pl.ds(NS * ZR, ZTAIL)])


_spmm = functools.partial(
    pl.kernel,
    out_type=jax.ShapeDtypeStruct((NC, N, D), jnp.float32),
    mesh=_mesh,
    scratch_types=[
        pltpu.VMEM((SB,), jnp.int32),
        pltpu.VMEM((SB,), jnp.float32),
        pltpu.VMEM((SBC, C), jnp.int32),
        pltpu.VMEM((C, D), jnp.float32),
        pltpu.VMEM((C, D), jnp.float32),
        pltpu.VMEM((C, D), jnp.float32),
        pltpu.VMEM_SHARED((N, D), jnp.float32),
        pltpu.SemaphoreType.DMA,
        pltpu.SemaphoreType.DMA,
        pltpu.SemaphoreType.DMA,
        pltpu.SemaphoreType.DMA,
        pltpu.SemaphoreType.DMA,
        pltpu.SemaphoreType.DMA,
    ],
)(_spmm_body)


def _dot(a, b):
    return jnp.dot(a, b, preferred_element_type=jnp.float32)


def _start_body(x_ref, w_ref, b_ref, tw_ref, tb_ref, sup_ref, gate_ref):
    x = x_ref[...]
    sup_ref[...] = _dot(x, w_ref[...]) + b_ref[...]
    gate_ref[...] = jax.nn.sigmoid(_dot(x, tw_ref[...]) + tb_ref[...])


def _bn_relu(hp0, hp1, gamma, beta):
    h = hp0 + hp1
    mu = jnp.mean(h, axis=0, keepdims=True)
    var = jnp.mean((h - mu) ** 2, axis=0, keepdims=True)
    hn = gamma * (h - mu) / jnp.sqrt(var + EPS) + beta
    return jnp.maximum(hn, 0.0)


def _advance_body(hp_ref, x_ref, gate_ref, gm_ref, bt_ref, w_ref, b_ref,
                  tw_ref, tb_ref, x_out, sup_out, gate_out):
    hr = _bn_relu(hp_ref[0], hp_ref[1], gm_ref[...], bt_ref[...])
    g = gate_ref[...]
    xn = g * hr + (1.0 - g) * x_ref[...]
    x_out[...] = xn
    sup_out[...] = _dot(xn, w_ref[...]) + b_ref[...]
    gate_out[...] = jax.nn.sigmoid(_dot(xn, tw_ref[...]) + tb_ref[...])


def _prelast_body(hp_ref, x_ref, gate_ref, gm_ref, bt_ref, w_ref, b_ref,
                  sup_ref):
    hr = _bn_relu(hp_ref[0], hp_ref[1], gm_ref[...], bt_ref[...])
    g = gate_ref[...]
    xn = g * hr + (1.0 - g) * x_ref[...]
    sup_ref[...] = _dot(xn, w_ref[...]) + b_ref[...]


def _final_body(hp_ref, w_ref, gl_ref, bl_ref, mb_ref, out_ref):
    hr = _bn_relu(hp_ref[0], hp_ref[1], gl_ref[0, 0], bl_ref[0, 0])
    s = jnp.sum(hr * w_ref[...], axis=0, keepdims=True)
    out_ref[...] = jax.nn.sigmoid(s[:, 0:1] + mb_ref[...])


def _sd(*shape):
    return jax.ShapeDtypeStruct(shape, jnp.float32)


_tc_start = pl.pallas_call(_start_body, out_shape=(_sd(N, D), _sd(N, D)))
_tc_advance = pl.pallas_call(
    _advance_body, out_shape=(_sd(N, D), _sd(N, D), _sd(N, D)))
_tc_prelast = pl.pallas_call(_prelast_body, out_shape=_sd(N, D))
_tc_final = pl.pallas_call(_final_body, out_shape=_sd(1, 1))


def kernel(vertices, adj_indices, adj_values, emb, gcn_W, gcn_b, gcn_W_last,
           gcn_b_last, bn_gamma, bn_beta, bn_gamma_last, bn_beta_last,
           te_W, te_b, mask_w, mask_b):
    verts = vertices.astype(jnp.int32)
    adj_idx = adj_indices.astype(jnp.int32)
    w_last = jnp.tile(gcn_W_last, (1, D))
    b_last = jnp.tile(gcn_b_last, (D,)).reshape(1, D)
    mask2d = jnp.tile(mask_w.reshape(-1, 1), (1, D))
    zeros_d = jnp.zeros((N, D), jnp.float32)

    x0, w2d = _emb_gather(verts, emb, mask2d)
    x = x0
    sup, gate = _tc_start(x0, gcn_W[0], gcn_b[0].reshape(1, D),
                          te_W[NHID - 2], te_b[NHID - 2].reshape(1, D))
    for i in range(NHID - 1):
        hp = _spmm(sup, adj_idx[i, 1],
                   adj_idx[i, 0].reshape(NW, SBN, SBC, C), adj_values[i],
                   zeros_d)
        gm = bn_gamma[i].reshape(1, D)
        bt = bn_beta[i].reshape(1, D)
        if i < NHID - 2:
            x, sup, gate = _tc_advance(
                hp, x, gate, gm, bt,
                gcn_W[i + 1], gcn_b[i + 1].reshape(1, D),
                te_W[i], te_b[i].reshape(1, D))
        else:
            sup = _tc_prelast(hp, x, gate, gm, bt, w_last, b_last)
    hp = _spmm(sup, adj_idx[NHID - 1, 1],
               adj_idx[NHID - 1, 0].reshape(NW, SBN, SBC, C),
               adj_values[NHID - 1], zeros_d)
    out = _tc_final(hp, w2d, bn_gamma_last.reshape(1, 1),
                    bn_beta_last.reshape(1, 1), mask_b.reshape(1, 1))
    return out.reshape(1)


# speedup vs baseline: 10.6603x; 1.0473x over previous
import functools

import numpy as np
import jax
import jax.numpy as jnp
from jax import lax
from jax.experimental import pallas as pl
from jax.experimental.pallas import tpu as pltpu
from jax.experimental.pallas import tpu_sc as plsc

N = 10000
D = 128
E = 320000
V = 100000
NHID = 7
EPS = 1e-5
NC, NS = 2, 16
NW = NC * NS
EPT = E // NW
C = 80
NCHUNK = EPT // C
SBN = 5
SBC = NCHUNK // SBN
SB = SBC * C
ZR = 624
ZTAIL = N - NS * ZR
GCH = N // 8

_HI = lax.Precision.HIGHEST
_DN = lax.GatherDimensionNumbers(
    offset_dims=(), collapsed_slice_dims=(0,), start_index_map=(0,))


def _bcast_lane(v16, j):
    return lax.gather(v16, jnp.full((16, 1), j, jnp.int32), _DN, (1,),
                      mode=lax.GatherScatterMode.PROMISE_IN_BOUNDS)
_mesh = plsc.VectorSubcoreMesh(core_axis_name="c", subcore_axis_name="s")


def _emb_body(verts, emb, mask2d, x0, w2d, idxb, rowb, mrowb, sem):
    tid = lax.axis_index("c") * NS + lax.axis_index("s")

    nmine = jnp.where(tid < GCH % NW, GCH // NW + 1, GCH // NW)

    def body(i, carry):
        base = pl.multiple_of((tid + i * NW) * 8, 8)
        pltpu.sync_copy(verts.at[pl.ds(base, 8)], idxb)
        pltpu.async_copy(emb.at[idxb], rowb, sem).wait()
        pltpu.sync_copy(rowb, x0.at[pl.ds(base, 8)])
        pltpu.async_copy(mask2d.at[idxb], mrowb, sem).wait()
        pltpu.sync_copy(mrowb, w2d.at[pl.ds(base, 8)])
        return carry

    lax.fori_loop(0, nmine, body, 0)


_emb_gather = functools.partial(
    pl.kernel,
    out_type=(jax.ShapeDtypeStruct((N, D), jnp.float32),
              jax.ShapeDtypeStruct((N, D), jnp.float32)),
    mesh=_mesh,
    scratch_types=[
        pltpu.VMEM((8,), jnp.int32),
        pltpu.VMEM((8, D), jnp.float32),
        pltpu.VMEM((8, D), jnp.float32),
        pltpu.SemaphoreType.DMA,
    ],
)(_emb_body)


def _spmm_body(support, src, dst4, vals, zeros, out,
               srcv, valv, dstv, rows0, rows1, rows2, rows3, accum,
               sg0, sg1, sg2, sg3, ss0, ss1, ss2, ss3):
    rowsb = (rows0, rows1, rows2, rows3)
    sg = (sg0, sg1, sg2, sg3)
    ss = (ss0, ss1, ss2, ss3)
    cid = lax.axis_index("c")
    sid = lax.axis_index("s")
    tid = cid * NS + sid
    rb = pl.multiple_of(sid * ZR, 8)
    pltpu.sync_copy(zeros.at[pl.ds(rb, ZR)], accum.at[pl.ds(rb, ZR)])

    @pl.when(sid == NS - 1)
    def _():
        pltpu.sync_copy(zeros.at[pl.ds(NS * ZR, ZTAIL)],
                        accum.at[pl.ds(NS * ZR, ZTAIL)])

    def gather_start(k, b):
        pltpu.async_copy(support.at[srcv.at[pl.ds(k * C, C)]], rowsb[b],
                         sg[b])

    def gather_wait(k, b):
        pltpu.make_async_copy(support.at[srcv.at[pl.ds(k * C, C)]],
                              rowsb[b], sg[b]).wait()

    def scale(k, b):
        r = rowsb[b]

        def grp(g, c2):
            v16 = valv[pl.ds(k * C + g * 16, 16)]
            for j in range(16):
                e = g * 16 + j
                vj = _bcast_lane(v16, j)
                for cb in range(D // 16):
                    sl = pl.ds(cb * 16, 16)
                    r[e, sl] = r[e, sl] * vj
            return c2

        lax.fori_loop(0, C // 16, grp, 0)

    def scatter_start(k, b):
        pltpu.async_copy(rowsb[b], accum.at[dstv.at[k]], ss[b], add=True)

    def scatter_wait(k, b):
        pltpu.make_async_copy(rowsb[b], accum.at[dstv.at[k]], ss[b]).wait()

    def step(k, b, b2, refill, drain):
        if refill:
            if drain:
                scatter_wait(k - 2, b2)
            gather_start(k + 2, b2)
        gather_wait(k, b)
        scale(k, b)
        scatter_start(k, b)

    def sblock(s, carry):
        base = pl.multiple_of(tid * EPT + s * SB, 8)
        pltpu.sync_copy(src.at[pl.ds(base, SB)], srcv)
        pltpu.sync_copy(vals.at[pl.ds(base, SB)], valv)
        pltpu.sync_copy(dst4.at[tid, s], dstv)
        gather_start(0, 0)
        gather_start(1, 1)
        step(0, 0, 2, True, False)
        step(1, 1, 3, True, False)

        def quad(t, c2):
            k = 2 + t * 4
            step(k, 2, 0, True, True)
            step(k + 1, 3, 1, True, True)
            step(k + 2, 0, 2, True, True)
            step(k + 3, 1, 3, True, True)
            return c2

        nsteady = (SBC - 5) // 4
        lax.fori_loop(0, nsteady, quad, 0)
        for k in range(2 + nsteady * 4, SBC):
            step(k, k % 4, (k + 2) % 4, k + 2 < SBC, True)
        for k in range(SBC - 4, SBC):
            scatter_wait(k, k % 4)
        return carry

    plsc.subcore_barrier()
    lax.fori_loop(0, SBN, sblock, 0)

    plsc.subcore_barrier()
    pltpu.sync_copy(accum.at[pl.ds(rb, ZR)], out.at[cid, pl.ds(rb, ZR)])

    @pl.when(sid == NS - 1)
    def _():
        pltpu.sync_copy(accum.at[pl.ds(NS * ZR, ZTAIL)],
                        out.at[cid, pl.ds(NS * ZR, ZTAIL)])


_spmm = functools.partial(
    pl.kernel,
    out_type=jax.ShapeDtypeStruct((NC, N, D), jnp.float32),
    mesh=_mesh,
    scratch_types=[
        pltpu.VMEM((SB,), jnp.int32),
        pltpu.VMEM((SB,), jnp.float32),
        pltpu.VMEM((SBC, C), jnp.int32),
        pltpu.VMEM((C, D), jnp.float32),
        pltpu.VMEM((C, D), jnp.float32),
        pltpu.VMEM((C, D), jnp.float32),
        pltpu.VMEM((C, D), jnp.float32),
        pltpu.VMEM_SHARED((N, D), jnp.float32),
        pltpu.SemaphoreType.DMA,
        pltpu.SemaphoreType.DMA,
        pltpu.SemaphoreType.DMA,
        pltpu.SemaphoreType.DMA,
        pltpu.SemaphoreType.DMA,
        pltpu.SemaphoreType.DMA,
        pltpu.SemaphoreType.DMA,
        pltpu.SemaphoreType.DMA,
    ],
)(_spmm_body)


def _dot(a, b):
    return jnp.dot(a, b, preferred_element_type=jnp.float32)


def _start_body(x_ref, w_ref, b_ref, tw_ref, tb_ref, sup_ref, gate_ref):
    x = x_ref[...]
    sup_ref[...] = _dot(x, w_ref[...]) + b_ref[...]
    gate_ref[...] = jax.nn.sigmoid(_dot(x, tw_ref[...]) + tb_ref[...])


def _bn_relu(hp0, hp1, gamma, beta):
    h = hp0 + hp1
    mu = jnp.mean(h, axis=0, keepdims=True)
    var = jnp.mean((h - mu) ** 2, axis=0, keepdims=True)
    hn = gamma * (h - mu) / jnp.sqrt(var + EPS) + beta
    return jnp.maximum(hn, 0.0)


def _advance_body(hp_ref, x_ref, gate_ref, gm_ref, bt_ref, w_ref, b_ref,
                  tw_ref, tb_ref, x_out, sup_out, gate_out):
    hr = _bn_relu(hp_ref[0], hp_ref[1], gm_ref[...], bt_ref[...])
    g = gate_ref[...]
    xn = g * hr + (1.0 - g) * x_ref[...]
    x_out[...] = xn
    sup_out[...] = _dot(xn, w_ref[...]) + b_ref[...]
    gate_out[...] = jax.nn.sigmoid(_dot(xn, tw_ref[...]) + tb_ref[...])


def _prelast_body(hp_ref, x_ref, gate_ref, gm_ref, bt_ref, w_ref, b_ref,
                  sup_ref):
    hr = _bn_relu(hp_ref[0], hp_ref[1], gm_ref[...], bt_ref[...])
    g = gate_ref[...]
    xn = g * hr + (1.0 - g) * x_ref[...]
    sup_ref[...] = _dot(xn, w_ref[...]) + b_ref[...]


def _final_body(hp_ref, w_ref, gl_ref, bl_ref, mb_ref, out_ref):
    hr = _bn_relu(hp_ref[0], hp_ref[1], gl_ref[0, 0], bl_ref[0, 0])
    s = jnp.sum(hr * w_ref[...], axis=0, keepdims=True)
    out_ref[...] = jax.nn.sigmoid(s[:, 0:1] + mb_ref[...])


def _sd(*shape):
    return jax.ShapeDtypeStruct(shape, jnp.float32)


_tc_start = pl.pallas_call(_start_body, out_shape=(_sd(N, D), _sd(N, D)))
_tc_advance = pl.pallas_call(
    _advance_body, out_shape=(_sd(N, D), _sd(N, D), _sd(N, D)))
_tc_prelast = pl.pallas_call(_prelast_body, out_shape=_sd(N, D))
_tc_final = pl.pallas_call(_final_body, out_shape=_sd(1, 1))


def kernel(vertices, adj_indices, adj_values, emb, gcn_W, gcn_b, gcn_W_last,
           gcn_b_last, bn_gamma, bn_beta, bn_gamma_last, bn_beta_last,
           te_W, te_b, mask_w, mask_b):
    verts = vertices.astype(jnp.int32)
    adj_idx = adj_indices.astype(jnp.int32)
    w_last = jnp.tile(gcn_W_last, (1, D))
    b_last = jnp.tile(gcn_b_last, (D,)).reshape(1, D)
    mask2d = jnp.tile(mask_w.reshape(-1, 1), (1, D))
    zeros_d = jnp.zeros((N, D), jnp.float32)

    x0, w2d = _emb_gather(verts, emb, mask2d)
    x = x0
    sup, gate = _tc_start(x0, gcn_W[0], gcn_b[0].reshape(1, D),
                          te_W[NHID - 2], te_b[NHID - 2].reshape(1, D))
    for i in range(NHID - 1):
        hp = _spmm(sup, adj_idx[i, 1],
                   adj_idx[i, 0].reshape(NW, SBN, SBC, C), adj_values[i],
                   zeros_d)
        gm = bn_gamma[i].reshape(1, D)
        bt = bn_beta[i].reshape(1, D)
        if i < NHID - 2:
            x, sup, gate = _tc_advance(
                hp, x, gate, gm, bt,
                gcn_W[i + 1], gcn_b[i + 1].reshape(1, D),
                te_W[i], te_b[i].reshape(1, D))
        else:
            sup = _tc_prelast(hp, x, gate, gm, bt, w_last, b_last)
    hp = _spmm(sup, adj_idx[NHID - 1, 1],
               adj_idx[NHID - 1, 0].reshape(NW, SBN, SBC, C),
               adj_values[NHID - 1], zeros_d)
    out = _tc_final(hp, w2d, bn_gamma_last.reshape(1, 1),
                    bn_beta_last.reshape(1, 1), mask_b.reshape(1, 1))
    return out.reshape(1)
